# Initial kernel scaffold; baseline (speedup 1.0000x reference)
import jax
import jax.numpy as jnp
from jax.experimental import pallas as pl


def kernel(pairs, W1, b1, w2, b2, mx, mm):
    raise NotImplementedError("write your pallas kernel here")


# speedup vs baseline: 4.8091x; 4.8091x over previous
import functools

import jax
import jax.numpy as jnp
from jax import lax
from jax.experimental import pallas as pl
from jax.experimental.pallas import tpu as pltpu
from jax.experimental.pallas import tpu_sc as plsc

H = 16
KM = 4
DIM = 2
IDX_CHUNK = 128
MW = 16


def _sc_gather(w1f, b1, w2, me, r1g, r2g):
    n_groups = r1g.shape[0]
    n = n_groups * IDX_CHUNK
    info = plsc.get_sparse_core_info()
    nc, ns = info.num_cores, info.num_subcores
    nw = nc * ns
    bpw = n // nw
    nch = bpw // IDX_CHUNK

    mesh = plsc.VectorSubcoreMesh(core_axis_name="c", subcore_axis_name="s")
    f32 = jnp.float32
    out_type = [
        jax.ShapeDtypeStruct((n, 2 * H), f32),
        jax.ShapeDtypeStruct((n, H), f32),
        jax.ShapeDtypeStruct((n, H), f32),
        jax.ShapeDtypeStruct((n, MW), f32),
        jax.ShapeDtypeStruct((n, 2 * H), f32),
        jax.ShapeDtypeStruct((n, H), f32),
        jax.ShapeDtypeStruct((n, H), f32),
        jax.ShapeDtypeStruct((n, MW), f32),
    ]
    scratch_types = [
        pltpu.VMEM((nch, IDX_CHUNK), jnp.int32),
        pltpu.VMEM((nch, IDX_CHUNK), jnp.int32),
        pltpu.VMEM((bpw, 2 * H), f32),
        pltpu.VMEM((bpw, H), f32),
        pltpu.VMEM((bpw, H), f32),
        pltpu.VMEM((bpw, MW), f32),
        pltpu.VMEM((bpw, 2 * H), f32),
        pltpu.VMEM((bpw, H), f32),
        pltpu.VMEM((bpw, H), f32),
        pltpu.VMEM((bpw, MW), f32),
        pltpu.SemaphoreType.DMA,
    ]

    @functools.partial(pl.kernel, mesh=mesh, out_type=out_type,
                       scratch_types=scratch_types,
                       compiler_params=pltpu.CompilerParams(
                           use_tc_tiling_on_sc=False))
    def k(tw1, tb1, tw2, tme, r1h, r2h,
          ow1a, ob1a, ow2a, omea, ow1b, ob1b, ow2b, omeb,
          i1, i2,
          bw1a, bb1a, bw2a, bmea, bw1b, bb1b, bw2b, bmeb, sem):
        wid = lax.axis_index("s") * nc + lax.axis_index("c")
        base = wid * bpw
        pltpu.sync_copy(r1h.at[pl.ds(wid * nch, nch), :], i1)
        pltpu.sync_copy(r2h.at[pl.ds(wid * nch, nch), :], i2)
        handles = []
        for idxv, bufs in ((i1, (bw1a, bb1a, bw2a, bmea)),
                           (i2, (bw1b, bb1b, bw2b, bmeb))):
            for tbl, buf in zip((tw1, tb1, tw2, tme), bufs):
                for c in range(nch):
                    handles.append(pltpu.async_copy(
                        tbl.at[idxv.at[c]],
                        buf.at[pl.ds(c * IDX_CHUNK, IDX_CHUNK), :],
                        sem))
        for hdl in handles:
            hdl.wait()
        outs = (ow1a, ob1a, ow2a, omea, ow1b, ob1b, ow2b, omeb)
        bufs = (bw1a, bb1a, bw2a, bmea, bw1b, bb1b, bw2b, bmeb)
        for buf, out in zip(bufs, outs):
            pltpu.sync_copy(buf, out.at[pl.ds(base, bpw), :])

    return k(w1f, b1, w2, me, r1g, r2g)


def _tc_body(w1a_r, b1a_r, w2a_r, mea_r, w1b_r, b1b_r, w2b_r, meb_r, out_r):
    def side(w1f, b1f, w2f, b2f, me_other):
        mxm = me_other[:, :2 * KM]
        mmm = me_other[:, 2 * KM:2 * KM + KM]
        va0 = w1f[:, :H]
        va1 = w1f[:, H:2 * H]
        u = jnp.zeros_like(b1f)
        for k in range(KM):
            x0 = mxm[:, 2 * k:2 * k + 1]
            x1 = mxm[:, 2 * k + 1:2 * k + 2]
            z = va0 * x0 + va1 * x1 + b1f
            u = u + mmm[:, k:k + 1] * jnp.tanh(z)
        s = jnp.sum(u * w2f, axis=1, keepdims=True)
        return s + b2f * jnp.sum(mmm, axis=1, keepdims=True)

    w1a, b1a, w2a, mea = w1a_r[...], b1a_r[...], w2a_r[...], mea_r[...]
    w1b, b1b, w2b, meb = w1b_r[...], b1b_r[...], w2b_r[...], meb_r[...]
    b2a = mea[:, 2 * KM + KM:2 * KM + KM + 1]
    b2b = meb[:, 2 * KM + KM:2 * KM + KM + 1]
    s1 = side(w1a, b1a, w2a, b2a, meb)
    s2 = side(w1b, b1b, w2b, b2b, mea)
    out_r[...] = (s1 + s2)[:, 0]


def _tc_compute(w1a, b1a, w2a, mea, w1b, b1b, w2b, meb):
    n = w1a.shape[0]
    bt = 4096
    widths = (2 * H, H, H, MW, 2 * H, H, H, MW)
    return pl.pallas_call(
        _tc_body,
        grid=(n // bt,),
        in_specs=[pl.BlockSpec((bt, w), lambda i: (i, 0)) for w in widths],
        out_specs=pl.BlockSpec((bt,), lambda i: (i,)),
        out_shape=jax.ShapeDtypeStruct((n,), jnp.float32),
    )(w1a, b1a, w2a, mea, w1b, b1b, w2b, meb)


def kernel(pairs, W1, b1, w2, b2, mx, mm):
    v = W1.shape[0]
    r1 = pairs[:, 0].astype(jnp.int32).reshape(-1, IDX_CHUNK)
    r2 = pairs[:, 1].astype(jnp.int32).reshape(-1, IDX_CHUNK)
    w1f = jnp.swapaxes(W1, 1, 2).reshape(v, 2 * H)
    me = jnp.concatenate(
        [mx.reshape(v, KM * DIM), mm, b2[:, None],
         jnp.zeros((v, MW - KM * DIM - KM - 1), jnp.float32)], axis=1)
    g = _sc_gather(w1f, b1, w2, me, r1, r2)
    return _tc_compute(*g)
